# Initial kernel scaffold; baseline (speedup 1.0000x reference)
import jax
import jax.numpy as jnp
from jax.experimental import pallas as pl


def kernel(lrsr_lv3, refsr_lv3, ref_lv1, ref_lv2, ref_lv3, params, gumbel_noise):
    raise NotImplementedError("write your pallas kernel here")


# speedup vs baseline: 33.0831x; 33.0831x over previous
import jax
import jax.numpy as jnp
from jax import lax
from jax.experimental import pallas as pl

_MAXK = 5
_HW = 1024


def _unfold(x, kernel, stride, pad):
    p = jax.lax.conv_general_dilated_patches(
        x, (kernel, kernel), (stride, stride), [(pad, pad), (pad, pad)])
    n, ck, ho, wo = p.shape
    return p.reshape(n, ck, ho * wo)


def _normalize(x, axis):
    nrm = jnp.sqrt(jnp.sum(x * x, axis=axis, keepdims=True))
    return x / jnp.maximum(nrm, 1e-12)


def _conv3x3(x, w, b):
    y = jax.lax.conv_general_dilated(
        x, w, (1, 1), [(1, 1), (1, 1)],
        dimension_numbers=('NCHW', 'OIHW', 'NCHW'))
    return y + b[None, :, None, None]


def _pos_encode(x, w, b, max_size=256):
    bsz, c, h, wd = x.shape
    lin = jnp.linspace(-1.0, 1.0, max_size)
    gx, gy = jnp.meshgrid(lin, lin, indexing='ij')
    grid = jnp.stack([gx, gy], axis=0)[None].astype(x.dtype)
    grid = jax.image.resize(grid, (1, 2, h, wd), method='bilinear')
    pos = _conv3x3(jnp.broadcast_to(grid, (bsz, 2, h, wd)), w, b)
    return x + pos


def _se(x, w1, b1, w2, b2):
    y = jnp.mean(x, axis=(2, 3))
    y = jnp.maximum(y @ w1.T + b1, 0.0)
    y = jax.nn.sigmoid(y @ w2.T + b2)
    return x * y[:, :, None, None]


_JT = 256


_JT = 256


def _topk_kernel(reft_ref, lr_ref, sc_ref, s3_ref):
    reft = reft_ref[0]
    lr = lr_ref[0]
    r = jnp.dot(reft, lr, preferred_element_type=jnp.float32)
    alpha = sc_ref[0, 0, 0]
    beta = sc_ref[0, 0, 1]
    cur = alpha * r + beta
    iota = lax.broadcasted_iota(jnp.int32, (_HW, _JT), 0)
    accs = [jnp.zeros((_HW, _JT), jnp.float32) for _ in range(3)]
    for i in range(_MAXK):
        m = jnp.max(cur, axis=0)
        im = jnp.min(jnp.where(cur == m[None, :], iota, jnp.int32(2 ** 30)),
                     axis=0)
        oh = iota == im[None, :]
        w = jax.nn.sigmoid(m)
        for lvl in range(3):
            kl = sc_ref[0, 0, 2 + lvl]
            wm = w * (jnp.float32(i) < kl).astype(jnp.float32)
            accs[lvl] = accs[lvl] + jnp.where(oh, wm[None, :], 0.0)
        if i + 1 < _MAXK:
            cur = jnp.where(iota == im[None, :], -jnp.inf, cur)
    for lvl in range(3):
        s3_ref[0, lvl] = accs[lvl]


def _run_topk(refsr_nrm_t, lrsr_nrm, scalars):
    n = refsr_nrm_t.shape[0]
    d = refsr_nrm_t.shape[2]
    return pl.pallas_call(
        _topk_kernel,
        grid=(n, _HW // _JT),
        in_specs=[
            pl.BlockSpec((1, _HW, d), lambda i, j: (i, 0, 0)),
            pl.BlockSpec((1, d, _JT), lambda i, j: (i, 0, j)),
            pl.BlockSpec((1, 1, 8), lambda i, j: (i, 0, 0)),
        ],
        out_specs=pl.BlockSpec((1, 3, _HW, _JT), lambda i, j: (i, 0, 0, j)),
        out_shape=jax.ShapeDtypeStruct((n, 3, _HW, _HW), jnp.float32),
    )(refsr_nrm_t, lrsr_nrm, scalars)


def _make_transfer_kernel(c, kk):
    inv = 1.0 / float(kk * kk)

    def _kernel(x_ref, s_ref, out_ref):
        y = x_ref[0, 0, 0]
        out_ref[0, 0, 0] = jnp.zeros((c, 34, 34), jnp.float32)
        for u in range(3):
            for v in range(3):
                a = y[:, u:u + 32, v:v + 32].reshape(c, _HW)
                t = jnp.dot(a, s_ref[0, 0], preferred_element_type=jnp.float32,
                            precision=lax.Precision.HIGHEST)
                cur = out_ref[0, 0, 0, :, u:u + 32, v:v + 32]
                out_ref[0, 0, 0, :, u:u + 32, v:v + 32] = (
                    cur + t.reshape(c, 32, 32) * inv)

    return _kernel


def _transfer_level(ref_feat, s3, lvl, kernel, stride, pad, pw, pb):
    n, c, hr, wr = ref_feat.shape
    s = stride
    ref_pos = _pos_encode(ref_feat, pw, pb)
    ref_pad = jnp.pad(ref_pos, ((0, 0), (0, 0), (pad, pad), (pad, pad)))
    xt = ref_pad.reshape(n, c, 34, s, 34, s).transpose(0, 3, 5, 1, 2, 4)
    out6 = pl.pallas_call(
        _make_transfer_kernel(c, kernel),
        grid=(n, s, s),
        in_specs=[
            pl.BlockSpec((1, 1, 1, c, 34, 34), lambda i, a, b: (i, a, b, 0, 0, 0)),
            pl.BlockSpec((1, 1, _HW, _HW), lambda i, a, b: (i, lvl, 0, 0)),
        ],
        out_specs=pl.BlockSpec((1, 1, 1, c, 34, 34),
                               lambda i, a, b: (i, a, b, 0, 0, 0)),
        out_shape=jax.ShapeDtypeStruct((n, s, s, c, 34, 34), jnp.float32),
    )(xt, s3)
    out_pad = out6.transpose(0, 3, 4, 1, 5, 2).reshape(n, c, 34 * s, 34 * s)
    hout = 32 * s
    return out_pad[:, :, pad:pad + hout, pad:pad + hout]


def kernel(lrsr_lv3, refsr_lv3, ref_lv1, ref_lv2, ref_lv3, params, gumbel_noise):
    n, c3, h, w = lrsr_lv3.shape
    k_probs = jax.nn.softmax(
        (params['k_logits'] + gumbel_noise) / params['temperature'], axis=-1)
    k_list = jnp.argmax(k_probs, axis=-1) + 1

    refsr_enc = _pos_encode(refsr_lv3, params['pos_w_lv3'], params['pos_b_lv3'])
    lrsr_unf = _unfold(lrsr_lv3, 3, 1, 1)
    refsr_unf = _unfold(refsr_enc, 3, 1, 1)
    refsr_nrm_t = _normalize(jnp.transpose(refsr_unf, (0, 2, 1)), 2)
    lrsr_nrm = _normalize(lrsr_unf, 1)

    ref_g = _normalize(jnp.mean(refsr_enc, axis=(2, 3)), 1)
    lr_g = _normalize(jnp.mean(lrsr_lv3, axis=(2, 3)), 1)
    rg = jnp.sum(ref_g * lr_g, axis=1)
    gw = params['global_weight']
    kf = k_list.astype(jnp.float32)
    scalars = jnp.concatenate([
        jnp.broadcast_to(1.0 - gw, (n, 1)),
        (gw * rg)[:, None],
        jnp.broadcast_to(kf[None, :], (n, 3)),
        jnp.zeros((n, 3), jnp.float32),
    ], axis=1).reshape(n, 1, 8)

    s3 = _run_topk(refsr_nrm_t, lrsr_nrm, scalars)

    outs = []
    levels = [
        (ref_lv3, 3, 1, 1, params['pos_w_lv3'], params['pos_b_lv3'], 0),
        (ref_lv2, 6, 2, 2, params['pos_w_lv2'], params['pos_b_lv2'], 1),
        (ref_lv1, 12, 4, 4, params['pos_w_lv1'], params['pos_b_lv1'], 2),
    ]
    for ref_feat, kk, st, pd, pw, pb, lvl in levels:
        outs.append(_transfer_level(ref_feat, s3, lvl, kk, st, pd, pw, pb))

    t_lv3 = _se(outs[0], params['se3_w1'], params['se3_b1'],
                params['se3_w2'], params['se3_b2'])
    t_lv2 = _se(outs[1], params['se2_w1'], params['se2_b1'],
                params['se2_w2'], params['se2_b2'])
    t_lv1 = _se(outs[2], params['se1_w1'], params['se1_b1'],
                params['se1_w2'], params['se1_b2'])
    return (t_lv3, t_lv2, t_lv1)


# speedup vs baseline: 53.1622x; 1.6069x over previous
import jax
import jax.numpy as jnp
from jax import lax
from jax.experimental import pallas as pl

_MAXK = 5
_HW = 1024


def _unfold(x, kernel, stride, pad):
    p = jax.lax.conv_general_dilated_patches(
        x, (kernel, kernel), (stride, stride), [(pad, pad), (pad, pad)])
    n, ck, ho, wo = p.shape
    return p.reshape(n, ck, ho * wo)


def _normalize(x, axis):
    nrm = jnp.sqrt(jnp.sum(x * x, axis=axis, keepdims=True))
    return x / jnp.maximum(nrm, 1e-12)


def _conv3x3(x, w, b):
    y = jax.lax.conv_general_dilated(
        x, w, (1, 1), [(1, 1), (1, 1)],
        dimension_numbers=('NCHW', 'OIHW', 'NCHW'))
    return y + b[None, :, None, None]


def _pos_encode(x, w, b, max_size=256):
    bsz, c, h, wd = x.shape
    lin = jnp.linspace(-1.0, 1.0, max_size)
    gx, gy = jnp.meshgrid(lin, lin, indexing='ij')
    grid = jnp.stack([gx, gy], axis=0)[None].astype(x.dtype)
    grid = jax.image.resize(grid, (1, 2, h, wd), method='bilinear')
    pos = _conv3x3(jnp.broadcast_to(grid, (bsz, 2, h, wd)), w, b)
    return x + pos


def _se(x, w1, b1, w2, b2):
    y = jnp.mean(x, axis=(2, 3))
    y = jnp.maximum(y @ w1.T + b1, 0.0)
    y = jax.nn.sigmoid(y @ w2.T + b2)
    return x * y[:, :, None, None]


_JT = 256


_JT = 256


def _topk_kernel(reft_ref, lr_ref, sc_ref, s3_ref):
    reft = reft_ref[0]
    lr = lr_ref[0]
    r = jnp.dot(reft, lr, preferred_element_type=jnp.float32)
    alpha = sc_ref[0, 0, 0]
    beta = sc_ref[0, 0, 1]
    cur = alpha * r + beta
    iota = lax.broadcasted_iota(jnp.int32, (_HW, _JT), 0)
    accs = [jnp.zeros((_HW, _JT), jnp.float32) for _ in range(3)]
    for i in range(_MAXK):
        m = jnp.max(cur, axis=0)
        im = jnp.min(jnp.where(cur == m[None, :], iota, jnp.int32(2 ** 30)),
                     axis=0)
        oh = iota == im[None, :]
        w = jax.nn.sigmoid(m)
        for lvl in range(3):
            kl = sc_ref[0, 0, 2 + lvl]
            wm = w * (jnp.float32(i) < kl).astype(jnp.float32)
            accs[lvl] = accs[lvl] + jnp.where(oh, wm[None, :], 0.0)
        if i + 1 < _MAXK:
            cur = jnp.where(iota == im[None, :], -jnp.inf, cur)
    for lvl in range(3):
        s3_ref[0, lvl] = accs[lvl]


def _run_topk(refsr_nrm_t, lrsr_nrm, scalars):
    n = refsr_nrm_t.shape[0]
    d = refsr_nrm_t.shape[2]
    return pl.pallas_call(
        _topk_kernel,
        grid=(n, _HW // _JT),
        in_specs=[
            pl.BlockSpec((1, _HW, d), lambda i, j: (i, 0, 0)),
            pl.BlockSpec((1, d, _JT), lambda i, j: (i, 0, j)),
            pl.BlockSpec((1, 1, 8), lambda i, j: (i, 0, 0)),
        ],
        out_specs=pl.BlockSpec((1, 3, _HW, _JT), lambda i, j: (i, 0, 0, j)),
        out_shape=jax.ShapeDtypeStruct((n, 3, _HW, _HW), jnp.float32),
    )(refsr_nrm_t, lrsr_nrm, scalars)


def _make_transfer_kernel(c, kk):
    inv = 1.0 / float(kk * kk)

    def _kernel(x_ref, s_ref, out_ref):
        y = x_ref[0, 0, 0]
        out_ref[0, 0, 0] = jnp.zeros((c, 34, 34), jnp.float32)
        for u in range(3):
            for v in range(3):
                a = y[:, u:u + 32, v:v + 32].reshape(c, _HW)
                t = jnp.dot(a, s_ref[0, 0], preferred_element_type=jnp.float32)
                cur = out_ref[0, 0, 0, :, u:u + 32, v:v + 32]
                out_ref[0, 0, 0, :, u:u + 32, v:v + 32] = (
                    cur + t.reshape(c, 32, 32) * inv)

    return _kernel


def _transfer_level(ref_feat, s3, lvl, kernel, stride, pad, pw, pb):
    n, c, hr, wr = ref_feat.shape
    s = stride
    ref_pos = _pos_encode(ref_feat, pw, pb)
    ref_pad = jnp.pad(ref_pos, ((0, 0), (0, 0), (pad, pad), (pad, pad)))
    xt = ref_pad.reshape(n, c, 34, s, 34, s).transpose(0, 3, 5, 1, 2, 4)
    out6 = pl.pallas_call(
        _make_transfer_kernel(c, kernel),
        grid=(n, s, s),
        in_specs=[
            pl.BlockSpec((1, 1, 1, c, 34, 34), lambda i, a, b: (i, a, b, 0, 0, 0)),
            pl.BlockSpec((1, 1, _HW, _HW), lambda i, a, b: (i, lvl, 0, 0)),
        ],
        out_specs=pl.BlockSpec((1, 1, 1, c, 34, 34),
                               lambda i, a, b: (i, a, b, 0, 0, 0)),
        out_shape=jax.ShapeDtypeStruct((n, s, s, c, 34, 34), jnp.float32),
    )(xt, s3)
    out_pad = out6.transpose(0, 3, 4, 1, 5, 2).reshape(n, c, 34 * s, 34 * s)
    hout = 32 * s
    return out_pad[:, :, pad:pad + hout, pad:pad + hout]


def kernel(lrsr_lv3, refsr_lv3, ref_lv1, ref_lv2, ref_lv3, params, gumbel_noise):
    n, c3, h, w = lrsr_lv3.shape
    k_probs = jax.nn.softmax(
        (params['k_logits'] + gumbel_noise) / params['temperature'], axis=-1)
    k_list = jnp.argmax(k_probs, axis=-1) + 1

    refsr_enc = _pos_encode(refsr_lv3, params['pos_w_lv3'], params['pos_b_lv3'])
    lrsr_unf = _unfold(lrsr_lv3, 3, 1, 1)
    refsr_unf = _unfold(refsr_enc, 3, 1, 1)
    refsr_nrm_t = _normalize(jnp.transpose(refsr_unf, (0, 2, 1)), 2)
    lrsr_nrm = _normalize(lrsr_unf, 1)

    ref_g = _normalize(jnp.mean(refsr_enc, axis=(2, 3)), 1)
    lr_g = _normalize(jnp.mean(lrsr_lv3, axis=(2, 3)), 1)
    rg = jnp.sum(ref_g * lr_g, axis=1)
    gw = params['global_weight']
    kf = k_list.astype(jnp.float32)
    scalars = jnp.concatenate([
        jnp.broadcast_to(1.0 - gw, (n, 1)),
        (gw * rg)[:, None],
        jnp.broadcast_to(kf[None, :], (n, 3)),
        jnp.zeros((n, 3), jnp.float32),
    ], axis=1).reshape(n, 1, 8)

    s3 = _run_topk(refsr_nrm_t, lrsr_nrm, scalars)

    outs = []
    levels = [
        (ref_lv3, 3, 1, 1, params['pos_w_lv3'], params['pos_b_lv3'], 0),
        (ref_lv2, 6, 2, 2, params['pos_w_lv2'], params['pos_b_lv2'], 1),
        (ref_lv1, 12, 4, 4, params['pos_w_lv1'], params['pos_b_lv1'], 2),
    ]
    for ref_feat, kk, st, pd, pw, pb, lvl in levels:
        outs.append(_transfer_level(ref_feat, s3, lvl, kk, st, pd, pw, pb))

    t_lv3 = _se(outs[0], params['se3_w1'], params['se3_b1'],
                params['se3_w2'], params['se3_b2'])
    t_lv2 = _se(outs[1], params['se2_w1'], params['se2_b1'],
                params['se2_w2'], params['se2_b2'])
    t_lv1 = _se(outs[2], params['se1_w1'], params['se1_b1'],
                params['se1_w2'], params['se1_b2'])
    return (t_lv3, t_lv2, t_lv1)


# speedup vs baseline: 55.6148x; 1.0461x over previous
import jax
import jax.numpy as jnp
from jax import lax
from jax.experimental import pallas as pl

_MAXK = 5
_HW = 1024


def _unfold(x, kernel, stride, pad):
    p = jax.lax.conv_general_dilated_patches(
        x, (kernel, kernel), (stride, stride), [(pad, pad), (pad, pad)])
    n, ck, ho, wo = p.shape
    return p.reshape(n, ck, ho * wo)


def _normalize(x, axis):
    nrm = jnp.sqrt(jnp.sum(x * x, axis=axis, keepdims=True))
    return x / jnp.maximum(nrm, 1e-12)


def _conv3x3(x, w, b):
    y = jax.lax.conv_general_dilated(
        x, w, (1, 1), [(1, 1), (1, 1)],
        dimension_numbers=('NCHW', 'OIHW', 'NCHW'))
    return y + b[None, :, None, None]


def _pos_encode(x, w, b, max_size=256):
    bsz, c, h, wd = x.shape
    lin = jnp.linspace(-1.0, 1.0, max_size)
    gx, gy = jnp.meshgrid(lin, lin, indexing='ij')
    grid = jnp.stack([gx, gy], axis=0)[None].astype(x.dtype)
    grid = jax.image.resize(grid, (1, 2, h, wd), method='bilinear')
    pos = _conv3x3(jnp.broadcast_to(grid, (bsz, 2, h, wd)), w, b)
    return x + pos


def _se(x, w1, b1, w2, b2):
    y = jnp.mean(x, axis=(2, 3))
    y = jnp.maximum(y @ w1.T + b1, 0.0)
    y = jax.nn.sigmoid(y @ w2.T + b2)
    return x * y[:, :, None, None]


_JT = 256


_JT = 256


def _topk_kernel(reft_ref, lr_ref, sc_ref, s3_ref):
    reft = reft_ref[0]
    lr = lr_ref[0]
    r = jnp.dot(reft, lr, preferred_element_type=jnp.float32)
    alpha = sc_ref[0, 0, 0]
    beta = sc_ref[0, 0, 1]
    cur = alpha * r + beta
    iota = lax.broadcasted_iota(jnp.int32, (_HW, _JT), 0)
    accs = [jnp.zeros((_HW, _JT), jnp.float32) for _ in range(3)]
    for i in range(_MAXK):
        m = jnp.max(cur, axis=0)
        im = jnp.min(jnp.where(cur == m[None, :], iota, jnp.int32(2 ** 30)),
                     axis=0)
        oh = iota == im[None, :]
        w = jax.nn.sigmoid(m)
        for lvl in range(3):
            kl = sc_ref[0, 0, 2 + lvl]
            wm = w * (jnp.float32(i) < kl).astype(jnp.float32)
            accs[lvl] = accs[lvl] + jnp.where(oh, wm[None, :], 0.0)
        if i + 1 < _MAXK:
            cur = jnp.where(iota == im[None, :], -jnp.inf, cur)
    for lvl in range(3):
        s3_ref[0, lvl] = accs[lvl]


def _run_topk(refsr_nrm_t, lrsr_nrm, scalars):
    n = refsr_nrm_t.shape[0]
    d = refsr_nrm_t.shape[2]
    return pl.pallas_call(
        _topk_kernel,
        grid=(n, _HW // _JT),
        in_specs=[
            pl.BlockSpec((1, _HW, d), lambda i, j: (i, 0, 0)),
            pl.BlockSpec((1, d, _JT), lambda i, j: (i, 0, j)),
            pl.BlockSpec((1, 1, 8), lambda i, j: (i, 0, 0)),
        ],
        out_specs=pl.BlockSpec((1, 3, _HW, _JT), lambda i, j: (i, 0, 0, j)),
        out_shape=jax.ShapeDtypeStruct((n, 3, _HW, _HW), jnp.float32),
    )(refsr_nrm_t, lrsr_nrm, scalars)


def _make_transfer_kernel(c, kk):
    inv = 1.0 / float(kk * kk)

    def _kernel(x_ref, s_ref, out_ref):
        y = x_ref[0, 0, 0]
        a = jnp.concatenate(
            [y[:, u:u + 32, v:v + 32].reshape(c, _HW)
             for u in range(3) for v in range(3)], axis=0)
        t = jnp.dot(a, s_ref[0, 0], preferred_element_type=jnp.float32)
        out_ref[0, 0, 0] = jnp.zeros((c, 34, 34), jnp.float32)
        for u in range(3):
            for v in range(3):
                blk = t[(u * 3 + v) * c:(u * 3 + v + 1) * c]
                cur = out_ref[0, 0, 0, :, u:u + 32, v:v + 32]
                out_ref[0, 0, 0, :, u:u + 32, v:v + 32] = (
                    cur + blk.reshape(c, 32, 32) * inv)

    return _kernel


def _transfer_level(ref_feat, s3, lvl, kernel, stride, pad, pw, pb):
    n, c, hr, wr = ref_feat.shape
    s = stride
    ref_pos = _pos_encode(ref_feat, pw, pb)
    ref_pad = jnp.pad(ref_pos, ((0, 0), (0, 0), (pad, pad), (pad, pad)))
    xt = ref_pad.reshape(n, c, 34, s, 34, s).transpose(0, 3, 5, 1, 2, 4)
    out6 = pl.pallas_call(
        _make_transfer_kernel(c, kernel),
        grid=(n, s, s),
        in_specs=[
            pl.BlockSpec((1, 1, 1, c, 34, 34), lambda i, a, b: (i, a, b, 0, 0, 0)),
            pl.BlockSpec((1, 1, _HW, _HW), lambda i, a, b: (i, lvl, 0, 0)),
        ],
        out_specs=pl.BlockSpec((1, 1, 1, c, 34, 34),
                               lambda i, a, b: (i, a, b, 0, 0, 0)),
        out_shape=jax.ShapeDtypeStruct((n, s, s, c, 34, 34), jnp.float32),
    )(xt, s3)
    out_pad = out6.transpose(0, 3, 4, 1, 5, 2).reshape(n, c, 34 * s, 34 * s)
    hout = 32 * s
    return out_pad[:, :, pad:pad + hout, pad:pad + hout]


def kernel(lrsr_lv3, refsr_lv3, ref_lv1, ref_lv2, ref_lv3, params, gumbel_noise):
    n, c3, h, w = lrsr_lv3.shape
    k_probs = jax.nn.softmax(
        (params['k_logits'] + gumbel_noise) / params['temperature'], axis=-1)
    k_list = jnp.argmax(k_probs, axis=-1) + 1

    refsr_enc = _pos_encode(refsr_lv3, params['pos_w_lv3'], params['pos_b_lv3'])
    lrsr_unf = _unfold(lrsr_lv3, 3, 1, 1)
    refsr_unf = _unfold(refsr_enc, 3, 1, 1)
    refsr_nrm_t = _normalize(jnp.transpose(refsr_unf, (0, 2, 1)), 2)
    lrsr_nrm = _normalize(lrsr_unf, 1)

    ref_g = _normalize(jnp.mean(refsr_enc, axis=(2, 3)), 1)
    lr_g = _normalize(jnp.mean(lrsr_lv3, axis=(2, 3)), 1)
    rg = jnp.sum(ref_g * lr_g, axis=1)
    gw = params['global_weight']
    kf = k_list.astype(jnp.float32)
    scalars = jnp.concatenate([
        jnp.broadcast_to(1.0 - gw, (n, 1)),
        (gw * rg)[:, None],
        jnp.broadcast_to(kf[None, :], (n, 3)),
        jnp.zeros((n, 3), jnp.float32),
    ], axis=1).reshape(n, 1, 8)

    s3 = _run_topk(refsr_nrm_t, lrsr_nrm, scalars)

    outs = []
    levels = [
        (ref_lv3, 3, 1, 1, params['pos_w_lv3'], params['pos_b_lv3'], 0),
        (ref_lv2, 6, 2, 2, params['pos_w_lv2'], params['pos_b_lv2'], 1),
        (ref_lv1, 12, 4, 4, params['pos_w_lv1'], params['pos_b_lv1'], 2),
    ]
    for ref_feat, kk, st, pd, pw, pb, lvl in levels:
        outs.append(_transfer_level(ref_feat, s3, lvl, kk, st, pd, pw, pb))

    t_lv3 = _se(outs[0], params['se3_w1'], params['se3_b1'],
                params['se3_w2'], params['se3_b2'])
    t_lv2 = _se(outs[1], params['se2_w1'], params['se2_b1'],
                params['se2_w2'], params['se2_b2'])
    t_lv1 = _se(outs[2], params['se1_w1'], params['se1_b1'],
                params['se1_w2'], params['se1_b2'])
    return (t_lv3, t_lv2, t_lv1)


# speedup vs baseline: 55.6772x; 1.0011x over previous
import jax
import jax.numpy as jnp
from jax import lax
from jax.experimental import pallas as pl

_MAXK = 5
_HW = 1024


def _unfold(x, kernel, stride, pad):
    p = jax.lax.conv_general_dilated_patches(
        x, (kernel, kernel), (stride, stride), [(pad, pad), (pad, pad)])
    n, ck, ho, wo = p.shape
    return p.reshape(n, ck, ho * wo)


def _normalize(x, axis):
    nrm = jnp.sqrt(jnp.sum(x * x, axis=axis, keepdims=True))
    return x / jnp.maximum(nrm, 1e-12)


def _conv3x3(x, w, b):
    y = jax.lax.conv_general_dilated(
        x, w, (1, 1), [(1, 1), (1, 1)],
        dimension_numbers=('NCHW', 'OIHW', 'NCHW'))
    return y + b[None, :, None, None]


def _pos_encode(x, w, b, max_size=256):
    bsz, c, h, wd = x.shape
    lin = jnp.linspace(-1.0, 1.0, max_size)
    gx, gy = jnp.meshgrid(lin, lin, indexing='ij')
    grid = jnp.stack([gx, gy], axis=0)[None].astype(x.dtype)
    grid = jax.image.resize(grid, (1, 2, h, wd), method='bilinear')
    pos = _conv3x3(jnp.broadcast_to(grid, (bsz, 2, h, wd)), w, b)
    return x + pos


def _se(x, w1, b1, w2, b2):
    y = jnp.mean(x, axis=(2, 3))
    y = jnp.maximum(y @ w1.T + b1, 0.0)
    y = jax.nn.sigmoid(y @ w2.T + b2)
    return x * y[:, :, None, None]


_JT = 256


_JT = 256


def _topk_kernel(reft_ref, lr_ref, sc_ref, s3_ref):
    reft = reft_ref[0]
    lr = lr_ref[0]
    r = jnp.dot(reft, lr, preferred_element_type=jnp.float32)
    alpha = sc_ref[0, 0, 0]
    beta = sc_ref[0, 0, 1]
    cur = alpha * r + beta
    iota = lax.broadcasted_iota(jnp.int32, (_HW, _JT), 0)
    accs = [jnp.zeros((_HW, _JT), jnp.float32) for _ in range(3)]
    for i in range(_MAXK):
        m = jnp.max(cur, axis=0)
        im = jnp.min(jnp.where(cur == m[None, :], iota, jnp.int32(2 ** 30)),
                     axis=0)
        oh = iota == im[None, :]
        w = jax.nn.sigmoid(m)
        for lvl in range(3):
            kl = sc_ref[0, 0, 2 + lvl]
            wm = w * (jnp.float32(i) < kl).astype(jnp.float32)
            accs[lvl] = accs[lvl] + jnp.where(oh, wm[None, :], 0.0)
        if i + 1 < _MAXK:
            cur = jnp.where(iota == im[None, :], -jnp.inf, cur)
    for lvl in range(3):
        s3_ref[0, lvl] = accs[lvl]


def _run_topk(refsr_nrm_t, lrsr_nrm, scalars):
    n = refsr_nrm_t.shape[0]
    d = refsr_nrm_t.shape[2]
    return pl.pallas_call(
        _topk_kernel,
        grid=(n, _HW // _JT),
        in_specs=[
            pl.BlockSpec((1, _HW, d), lambda i, j: (i, 0, 0)),
            pl.BlockSpec((1, d, _JT), lambda i, j: (i, 0, j)),
            pl.BlockSpec((1, 1, 8), lambda i, j: (i, 0, 0)),
        ],
        out_specs=pl.BlockSpec((1, 3, _HW, _JT), lambda i, j: (i, 0, 0, j)),
        out_shape=jax.ShapeDtypeStruct((n, 3, _HW, _HW), jnp.float32),
    )(refsr_nrm_t, lrsr_nrm, scalars)


def _make_transfer_kernel(c, kk):
    inv = 1.0 / float(kk * kk)

    def _kernel(x_ref, s_ref, out_ref):
        y = x_ref[0, :, 0, 0]
        a = jnp.concatenate(
            [y[:, u:u + 32, v:v + 32].reshape(c, _HW)
             for u in range(3) for v in range(3)], axis=0)
        t = jnp.dot(a, s_ref[0, 0], preferred_element_type=jnp.float32)
        out_ref[0, 0, 0] = jnp.zeros((c, 34, 34), jnp.float32)
        for u in range(3):
            for v in range(3):
                blk = t[(u * 3 + v) * c:(u * 3 + v + 1) * c]
                cur = out_ref[0, 0, 0, :, u:u + 32, v:v + 32]
                out_ref[0, 0, 0, :, u:u + 32, v:v + 32] = (
                    cur + blk.reshape(c, 32, 32) * inv)

    return _kernel


def _transfer_level(ref_feat, s3, lvl, kernel, stride, pad, pw, pb):
    n, c, hr, wr = ref_feat.shape
    s = stride
    ref_pos = _pos_encode(ref_feat, pw, pb)
    ref_pad = jnp.pad(ref_pos, ((0, 0), (0, 0), (pad, pad), (pad, pad)))
    if s > 1:
        xt = jax.lax.conv_general_dilated_patches(
            ref_pad, (s, s), (s, s), [(0, 0), (0, 0)])
    else:
        xt = ref_pad
    xt = xt.reshape(n, c, s, s, 34, 34)
    out6 = pl.pallas_call(
        _make_transfer_kernel(c, kernel),
        grid=(n, s, s),
        in_specs=[
            pl.BlockSpec((1, c, 1, 1, 34, 34), lambda i, a, b: (i, 0, a, b, 0, 0)),
            pl.BlockSpec((1, 1, _HW, _HW), lambda i, a, b: (i, lvl, 0, 0)),
        ],
        out_specs=pl.BlockSpec((1, 1, 1, c, 34, 34),
                               lambda i, a, b: (i, a, b, 0, 0, 0)),
        out_shape=jax.ShapeDtypeStruct((n, s, s, c, 34, 34), jnp.float32),
    )(xt, s3)
    out_pad = out6.transpose(0, 3, 4, 1, 5, 2).reshape(n, c, 34 * s, 34 * s)
    hout = 32 * s
    return out_pad[:, :, pad:pad + hout, pad:pad + hout]


def kernel(lrsr_lv3, refsr_lv3, ref_lv1, ref_lv2, ref_lv3, params, gumbel_noise):
    n, c3, h, w = lrsr_lv3.shape
    k_probs = jax.nn.softmax(
        (params['k_logits'] + gumbel_noise) / params['temperature'], axis=-1)
    k_list = jnp.argmax(k_probs, axis=-1) + 1

    refsr_enc = _pos_encode(refsr_lv3, params['pos_w_lv3'], params['pos_b_lv3'])
    lrsr_unf = _unfold(lrsr_lv3, 3, 1, 1)
    refsr_unf = _unfold(refsr_enc, 3, 1, 1)
    refsr_nrm_t = _normalize(jnp.transpose(refsr_unf, (0, 2, 1)), 2)
    lrsr_nrm = _normalize(lrsr_unf, 1)

    ref_g = _normalize(jnp.mean(refsr_enc, axis=(2, 3)), 1)
    lr_g = _normalize(jnp.mean(lrsr_lv3, axis=(2, 3)), 1)
    rg = jnp.sum(ref_g * lr_g, axis=1)
    gw = params['global_weight']
    kf = k_list.astype(jnp.float32)
    scalars = jnp.concatenate([
        jnp.broadcast_to(1.0 - gw, (n, 1)),
        (gw * rg)[:, None],
        jnp.broadcast_to(kf[None, :], (n, 3)),
        jnp.zeros((n, 3), jnp.float32),
    ], axis=1).reshape(n, 1, 8)

    s3 = _run_topk(refsr_nrm_t, lrsr_nrm, scalars)

    outs = []
    levels = [
        (ref_lv3, 3, 1, 1, params['pos_w_lv3'], params['pos_b_lv3'], 0),
        (ref_lv2, 6, 2, 2, params['pos_w_lv2'], params['pos_b_lv2'], 1),
        (ref_lv1, 12, 4, 4, params['pos_w_lv1'], params['pos_b_lv1'], 2),
    ]
    for ref_feat, kk, st, pd, pw, pb, lvl in levels:
        outs.append(_transfer_level(ref_feat, s3, lvl, kk, st, pd, pw, pb))

    t_lv3 = _se(outs[0], params['se3_w1'], params['se3_b1'],
                params['se3_w2'], params['se3_b2'])
    t_lv2 = _se(outs[1], params['se2_w1'], params['se2_b1'],
                params['se2_w2'], params['se2_b2'])
    t_lv1 = _se(outs[2], params['se1_w1'], params['se1_b1'],
                params['se1_w2'], params['se1_b2'])
    return (t_lv3, t_lv2, t_lv1)


# speedup vs baseline: 56.4057x; 1.0131x over previous
import jax
import jax.numpy as jnp
from jax import lax
from jax.experimental import pallas as pl

_MAXK = 5
_HW = 1024


def _unfold(x, kernel, stride, pad):
    p = jax.lax.conv_general_dilated_patches(
        x, (kernel, kernel), (stride, stride), [(pad, pad), (pad, pad)])
    n, ck, ho, wo = p.shape
    return p.reshape(n, ck, ho * wo)


def _normalize(x, axis):
    nrm = jnp.sqrt(jnp.sum(x * x, axis=axis, keepdims=True))
    return x / jnp.maximum(nrm, 1e-12)


def _conv3x3(x, w, b):
    y = jax.lax.conv_general_dilated(
        x, w, (1, 1), [(1, 1), (1, 1)],
        dimension_numbers=('NCHW', 'OIHW', 'NCHW'))
    return y + b[None, :, None, None]


def _pos_encode(x, w, b, max_size=256):
    bsz, c, h, wd = x.shape
    lin = jnp.linspace(-1.0, 1.0, max_size)
    gx, gy = jnp.meshgrid(lin, lin, indexing='ij')
    grid = jnp.stack([gx, gy], axis=0)[None].astype(x.dtype)
    grid = jax.image.resize(grid, (1, 2, h, wd), method='bilinear')
    pos = _conv3x3(jnp.broadcast_to(grid, (bsz, 2, h, wd)), w, b)
    return x + pos


def _se(x, w1, b1, w2, b2):
    y = jnp.mean(x, axis=(2, 3))
    y = jnp.maximum(y @ w1.T + b1, 0.0)
    y = jax.nn.sigmoid(y @ w2.T + b2)
    return x * y[:, :, None, None]


_JT = 256


_JT = 256


def _topk_kernel(reft_ref, lr_ref, sc_ref, s3_ref):
    reft = reft_ref[0]
    lr = lr_ref[0]
    r = jnp.dot(reft, lr, preferred_element_type=jnp.float32)
    alpha = sc_ref[0, 0, 0]
    beta = sc_ref[0, 0, 1]
    cur = alpha * r + beta
    iota = lax.broadcasted_iota(jnp.int32, (_HW, _JT), 0)
    accs = [jnp.zeros((_HW, _JT), jnp.float32) for _ in range(3)]
    for i in range(_MAXK):
        m = jnp.max(cur, axis=0)
        im = jnp.min(jnp.where(cur == m[None, :], iota, jnp.int32(2 ** 30)),
                     axis=0)
        oh = iota == im[None, :]
        w = jax.nn.sigmoid(m)
        for lvl in range(3):
            kl = sc_ref[0, 0, 2 + lvl]
            wm = w * (jnp.float32(i) < kl).astype(jnp.float32)
            accs[lvl] = accs[lvl] + jnp.where(oh, wm[None, :], 0.0)
        if i + 1 < _MAXK:
            cur = jnp.where(iota == im[None, :], -jnp.inf, cur)
    for lvl in range(3):
        s3_ref[0, lvl] = accs[lvl]


def _run_topk(refsr_nrm_t, lrsr_nrm, scalars):
    n = refsr_nrm_t.shape[0]
    d = refsr_nrm_t.shape[2]
    return pl.pallas_call(
        _topk_kernel,
        grid=(n, _HW // _JT),
        in_specs=[
            pl.BlockSpec((1, _HW, d), lambda i, j: (i, 0, 0)),
            pl.BlockSpec((1, d, _JT), lambda i, j: (i, 0, j)),
            pl.BlockSpec((1, 1, 8), lambda i, j: (i, 0, 0)),
        ],
        out_specs=pl.BlockSpec((1, 3, _HW, _JT), lambda i, j: (i, 0, 0, j)),
        out_shape=jax.ShapeDtypeStruct((n, 3, _HW, _HW), jnp.float32),
    )(refsr_nrm_t, lrsr_nrm, scalars)


def _make_transfer_kernel(c, kk):
    inv = 1.0 / float(kk * kk)

    def _kernel(x_ref, s_ref, out_ref):
        y = x_ref[0, 0, 0]
        a = jnp.concatenate(
            [y[:, u:u + 32, v:v + 32].reshape(c, _HW)
             for u in range(3) for v in range(3)], axis=0)
        t = jnp.dot(a, s_ref[0, 0], preferred_element_type=jnp.float32)
        out_ref[0, 0, 0] = jnp.zeros((c, 34, 34), jnp.float32)
        for u in range(3):
            for v in range(3):
                blk = t[(u * 3 + v) * c:(u * 3 + v + 1) * c]
                cur = out_ref[0, 0, 0, :, u:u + 32, v:v + 32]
                out_ref[0, 0, 0, :, u:u + 32, v:v + 32] = (
                    cur + blk.reshape(c, 32, 32) * inv)

    return _kernel


def _transfer_level(ref_feat, s3, lvl, kernel, stride, pad, pw, pb):
    n, c, hr, wr = ref_feat.shape
    s = stride
    ref_pos = _pos_encode(ref_feat, pw, pb)
    ref_pad = jnp.pad(ref_pos, ((0, 0), (0, 0), (pad, pad), (pad, pad)))
    xt = ref_pad.reshape(n, c, 34, s, 34, s).transpose(0, 3, 5, 1, 2, 4)
    out6 = pl.pallas_call(
        _make_transfer_kernel(c, kernel),
        grid=(n, s, s),
        in_specs=[
            pl.BlockSpec((1, 1, 1, c, 34, 34), lambda i, a, b: (i, a, b, 0, 0, 0)),
            pl.BlockSpec((1, 1, _HW, _HW), lambda i, a, b: (i, lvl, 0, 0)),
        ],
        out_specs=pl.BlockSpec((1, 1, 1, c, 34, 34),
                               lambda i, a, b: (i, a, b, 0, 0, 0)),
        out_shape=jax.ShapeDtypeStruct((n, s, s, c, 34, 34), jnp.float32),
    )(xt, s3)
    out_pad = out6.transpose(0, 3, 4, 1, 5, 2).reshape(n, c, 34 * s, 34 * s)
    hout = 32 * s
    return out_pad[:, :, pad:pad + hout, pad:pad + hout]


def kernel(lrsr_lv3, refsr_lv3, ref_lv1, ref_lv2, ref_lv3, params, gumbel_noise):
    n, c3, h, w = lrsr_lv3.shape
    k_probs = jax.nn.softmax(
        (params['k_logits'] + gumbel_noise) / params['temperature'], axis=-1)
    k_list = jnp.argmax(k_probs, axis=-1) + 1

    refsr_enc = _pos_encode(refsr_lv3, params['pos_w_lv3'], params['pos_b_lv3'])
    lrsr_unf = _unfold(lrsr_lv3, 3, 1, 1)
    refsr_unf = _unfold(refsr_enc, 3, 1, 1)
    refsr_nrm_t = _normalize(jnp.transpose(refsr_unf, (0, 2, 1)), 2)
    lrsr_nrm = _normalize(lrsr_unf, 1)

    ref_g = _normalize(jnp.mean(refsr_enc, axis=(2, 3)), 1)
    lr_g = _normalize(jnp.mean(lrsr_lv3, axis=(2, 3)), 1)
    rg = jnp.sum(ref_g * lr_g, axis=1)
    gw = params['global_weight']
    kf = k_list.astype(jnp.float32)
    scalars = jnp.concatenate([
        jnp.broadcast_to(1.0 - gw, (n, 1)),
        (gw * rg)[:, None],
        jnp.broadcast_to(kf[None, :], (n, 3)),
        jnp.zeros((n, 3), jnp.float32),
    ], axis=1).reshape(n, 1, 8)

    s3 = _run_topk(refsr_nrm_t, lrsr_nrm, scalars)

    outs = []
    levels = [
        (ref_lv3, 3, 1, 1, params['pos_w_lv3'], params['pos_b_lv3'], 0),
        (ref_lv2, 6, 2, 2, params['pos_w_lv2'], params['pos_b_lv2'], 1),
        (ref_lv1, 12, 4, 4, params['pos_w_lv1'], params['pos_b_lv1'], 2),
    ]
    for ref_feat, kk, st, pd, pw, pb, lvl in levels:
        outs.append(_transfer_level(ref_feat, s3, lvl, kk, st, pd, pw, pb))

    t_lv3 = _se(outs[0], params['se3_w1'], params['se3_b1'],
                params['se3_w2'], params['se3_b2'])
    t_lv2 = _se(outs[1], params['se2_w1'], params['se2_b1'],
                params['se2_w2'], params['se2_b2'])
    t_lv1 = _se(outs[2], params['se1_w1'], params['se1_b1'],
                params['se1_w2'], params['se1_b2'])
    return (t_lv3, t_lv2, t_lv1)
